# Initial kernel scaffold; baseline (speedup 1.0000x reference)
import jax
import jax.numpy as jnp
from jax.experimental import pallas as pl


def kernel(x, edge_index, edge_attr, W1, b1, W2, b2, W3, b3, Wl, bl):
    raise NotImplementedError("write your pallas kernel here")


# speedup vs baseline: 16.3296x; 16.3296x over previous
import functools

import jax
import jax.numpy as jnp
from jax import lax
from jax.experimental import pallas as pl
from jax.experimental.pallas import tpu as pltpu
from jax.experimental.pallas import tpu_sc as plsc

NC = 2
NS = 16
NW = NC * NS
K = 80


def _sc_mesh():
    return plsc.VectorSubcoreMesh(core_axis_name="c", subcore_axis_name="s")


@functools.lru_cache(maxsize=None)
def _make_deg_kernel(E, N):
    e_per = E // NW
    n_chunks = e_per // K

    @functools.partial(
        pl.kernel,
        out_type=jax.ShapeDtypeStruct((NC, 1, N), jnp.float32),
        mesh=_sc_mesh(),
        scratch_types=[
            pltpu.VMEM((e_per,), jnp.int32),
            pltpu.VMEM((K,), jnp.int32),
            pltpu.VMEM((K,), jnp.float32),
            pltpu.VMEM_SHARED((N,), jnp.float32),
        ],
    )
    def deg_kernel(col_hbm, zeros_n_hbm, degp_hbm, col_all, col_buf, ones_buf, deg_sh):
        c = lax.axis_index("c")
        s = lax.axis_index("s")
        w = c * NS + s

        @pl.when(s == 0)
        def _():
            pltpu.sync_copy(zeros_n_hbm, deg_sh)

        pltpu.sync_copy(col_hbm.at[pl.ds(w * e_per, e_per)], col_all)
        for k in range(K // 16):
            ones_buf[pl.ds(k * 16, 16)] = jnp.ones((16,), jnp.float32)
        plsc.subcore_barrier()

        @pl.loop(0, n_chunks)
        def _(i):
            off = i * K
            for k in range(K // 16):
                col_buf[pl.ds(k * 16, 16)] = col_all[pl.ds(off + k * 16, 16)]
            pltpu.sync_copy(ones_buf, deg_sh.at[col_buf], add=True)

        plsc.subcore_barrier()

        @pl.when(s == 0)
        def _():
            pltpu.sync_copy(deg_sh, degp_hbm.at[c, 0])

    return deg_kernel


@functools.lru_cache(maxsize=None)
def _make_scatter_kernel(E, N, H, with_q):
    e_per = E // NW
    n_chunks = e_per // K
    Np = -(-N // (8 * NS)) * (8 * NS)
    n_per = Np // NS

    out_type = [jax.ShapeDtypeStruct((NC, Np, H), jnp.float32)]
    scratch = [
        pltpu.VMEM((e_per,), jnp.int32),
        pltpu.VMEM((e_per,), jnp.int32),
        pltpu.VMEM((K,), jnp.int32),
        pltpu.VMEM((K,), jnp.int32),
        pltpu.VMEM((K, H), jnp.float32),
        pltpu.VMEM_SHARED((Np, H), jnp.float32),
        pltpu.SemaphoreType.DMA,
    ]
    if with_q:
        out_type.append(jax.ShapeDtypeStruct((NC, 1, N), jnp.float32))
        scratch += [
            pltpu.VMEM((K,), jnp.float32),
            pltpu.VMEM_SHARED((N,), jnp.float32),
            pltpu.SemaphoreType.DMA,
        ]

    def body(*refs):
        if with_q:
            (y_hbm, row_hbm, col_hbm, dis_hbm, zeros_nh, zeros_n,
             vout, qout,
             row_all, col_all, row_buf, col_buf, rows_v, acc_sh, sem,
             dvals, q_sh, sem2) = refs
        else:
            (y_hbm, row_hbm, col_hbm, zeros_nh,
             vout,
             row_all, col_all, row_buf, col_buf, rows_v, acc_sh, sem) = refs

        c = lax.axis_index("c")
        s = lax.axis_index("s")
        w = c * NS + s

        pltpu.sync_copy(zeros_nh.at[pl.ds(s * n_per, n_per)],
                        acc_sh.at[pl.ds(s * n_per, n_per)])
        if with_q:
            @pl.when(s == 0)
            def _():
                pltpu.sync_copy(zeros_n, q_sh)
        pltpu.sync_copy(row_hbm.at[pl.ds(w * e_per, e_per)], row_all)
        pltpu.sync_copy(col_hbm.at[pl.ds(w * e_per, e_per)], col_all)
        plsc.subcore_barrier()

        @pl.loop(0, n_chunks)
        def _(i):
            off = i * K
            for k in range(K // 16):
                row_buf[pl.ds(k * 16, 16)] = row_all[pl.ds(off + k * 16, 16)]
                col_buf[pl.ds(k * 16, 16)] = col_all[pl.ds(off + k * 16, 16)]
            pltpu.async_copy(y_hbm.at[row_buf], rows_v, sem).wait()
            pltpu.sync_copy(rows_v, acc_sh.at[col_buf], add=True)
            if with_q:
                pltpu.async_copy(dis_hbm.at[col_buf], dvals, sem2).wait()
                pltpu.sync_copy(dvals, q_sh.at[row_buf], add=True)

        plsc.subcore_barrier()

        pltpu.sync_copy(acc_sh.at[pl.ds(s * n_per, n_per)],
                        vout.at[c, pl.ds(s * n_per, n_per)])
        if with_q:
            @pl.when(s == 0)
            def _():
                pltpu.sync_copy(q_sh, qout.at[c, 0])

    return pl.kernel(
        body,
        out_type=tuple(out_type) if with_q else out_type[0],
        mesh=_sc_mesh(),
        scratch_types=scratch,
    )


def _tc_dis_y1(degp, x, W1, BN=1000):
    N, D = x.shape
    H = W1.shape[1]

    def tc1(degp_ref, x_ref, w_ref, dis_ref, y_ref):
        dp = degp_ref[...]
        deg = dp[0] + dp[1]
        dis = jnp.where(deg > 0, lax.rsqrt(deg), 0.0)
        dis_ref[...] = dis
        xw = jnp.dot(x_ref[...], w_ref[...], preferred_element_type=jnp.float32)
        y_ref[...] = dis * xw

    return pl.pallas_call(
        tc1,
        grid=(N // BN,),
        in_specs=[
            pl.BlockSpec((NC, BN, 1), lambda i: (0, i, 0)),
            pl.BlockSpec((BN, D), lambda i: (i, 0)),
            pl.BlockSpec((D, H), lambda i: (0, 0)),
        ],
        out_specs=[
            pl.BlockSpec((BN, 1), lambda i: (i, 0)),
            pl.BlockSpec((BN, H), lambda i: (i, 0)),
        ],
        out_shape=[
            jax.ShapeDtypeStruct((N, 1), jnp.float32),
            jax.ShapeDtypeStruct((N, H), jnp.float32),
        ],
    )(degp, x, W1)


def _tc_mid(vp, dis, b, W, BN=1000):
    _, _, H = vp.shape
    N = dis.shape[0]

    def tc2(vp_ref, dis_ref, b_ref, w_ref, y_ref):
        v = vp_ref[0] + vp_ref[1]
        dis = dis_ref[...]
        h = dis * v + b_ref[...]
        y_ref[...] = dis * jnp.dot(
            h, w_ref[...], preferred_element_type=jnp.float32)

    return pl.pallas_call(
        tc2,
        grid=(N // BN,),
        in_specs=[
            pl.BlockSpec((NC, BN, H), lambda i: (0, i, 0)),
            pl.BlockSpec((BN, 1), lambda i: (i, 0)),
            pl.BlockSpec((H,), lambda i: (0,)),
            pl.BlockSpec((H, H), lambda i: (0, 0)),
        ],
        out_specs=pl.BlockSpec((BN, H), lambda i: (i, 0)),
        out_shape=jax.ShapeDtypeStruct((N, H), jnp.float32),
    )(vp, dis, b, W)


def _tc_final(vp, qp, dis, b2, W3, Wl, b3, bl, BN=1000):
    _, _, H = vp.shape
    N = dis.shape[0]
    nsteps = N // BN

    def tc3(vp_ref, qp_ref, dis_ref, b2_ref, w3_ref, wl_ref, b3_ref, bl_ref,
            out_ref, sacc):
        i = pl.program_id(0)

        @pl.when(i == 0)
        def _():
            sacc[...] = jnp.zeros_like(sacc)

        v = vp_ref[0] + vp_ref[1]
        dis = dis_ref[...]
        h2 = dis * v + b2_ref[...]
        g = dis * (qp_ref[0] + qp_ref[1])
        m = jnp.dot(h2, w3_ref[...], preferred_element_type=jnp.float32)
        sacc[...] += jnp.sum(g * m, axis=0, keepdims=True)

        @pl.when(i == nsteps - 1)
        def _():
            pooled = sacc[...] * (1.0 / N) + b3_ref[...][None, :]
            out_ref[...] = jnp.dot(pooled, wl_ref[...],
                                   preferred_element_type=jnp.float32) + bl_ref[...]

    return pl.pallas_call(
        tc3,
        grid=(nsteps,),
        in_specs=[
            pl.BlockSpec((NC, BN, H), lambda i: (0, i, 0)),
            pl.BlockSpec((NC, BN, 1), lambda i: (0, i, 0)),
            pl.BlockSpec((BN, 1), lambda i: (i, 0)),
            pl.BlockSpec((H,), lambda i: (0,)),
            pl.BlockSpec((H, H), lambda i: (0, 0)),
            pl.BlockSpec((H, 1), lambda i: (0, 0)),
            pl.BlockSpec((H,), lambda i: (0,)),
            pl.BlockSpec((1,), lambda i: (0,)),
        ],
        out_specs=pl.BlockSpec((1, 1), lambda i: (0, 0)),
        out_shape=jax.ShapeDtypeStruct((1, 1), jnp.float32),
        scratch_shapes=[pltpu.VMEM((1, H), jnp.float32)],
    )(vp, qp, dis, b2, W3, Wl, b3, bl)


def kernel(x, edge_index, edge_attr, W1, b1, W2, b2, W3, b3, Wl, bl):
    N, _ = x.shape
    H = W1.shape[1]
    E = edge_index.shape[1]
    row = edge_index[0]
    col = edge_index[1]
    Np = -(-N // (8 * NS)) * (8 * NS)
    zeros_nh = jnp.zeros((Np, H), jnp.float32)
    zeros_n = jnp.zeros((N,), jnp.float32)

    degp = _make_deg_kernel(E, N)(col, zeros_n)
    dis2, y1 = _tc_dis_y1(degp.reshape(NC, N, 1), x, W1)
    dis1 = dis2.reshape(N)
    v1p, qp = _make_scatter_kernel(E, N, H, True)(
        y1, row, col, dis1, zeros_nh, zeros_n)
    y2 = _tc_mid(v1p, dis2, b1, W2)
    v2p = _make_scatter_kernel(E, N, H, False)(y2, row, col, zeros_nh)
    return _tc_final(v2p, qp.reshape(NC, N, 1), dis2, b2, W3, Wl, b3, bl)


# speedup vs baseline: 23.5814x; 1.4441x over previous
import functools

import jax
import jax.numpy as jnp
from jax import lax
from jax.experimental import pallas as pl
from jax.experimental.pallas import tpu as pltpu
from jax.experimental.pallas import tpu_sc as plsc

NC = 2
NS = 16
NW = NC * NS
K = 80


def _sc_mesh():
    return plsc.VectorSubcoreMesh(core_axis_name="c", subcore_axis_name="s")


@functools.lru_cache(maxsize=None)
def _make_deg_kernel(E, N):
    e_per = E // NW
    n_chunks = e_per // K

    @functools.partial(
        pl.kernel,
        out_type=jax.ShapeDtypeStruct((NC, 1, N), jnp.float32),
        mesh=_sc_mesh(),
        scratch_types=[
            pltpu.VMEM((e_per,), jnp.int32),
            pltpu.VMEM((K,), jnp.int32),
            pltpu.VMEM((K,), jnp.float32),
            pltpu.VMEM_SHARED((N,), jnp.float32),
        ],
    )
    def deg_kernel(col_hbm, zeros_n_hbm, degp_hbm, col_all, col_buf, ones_buf, deg_sh):
        c = lax.axis_index("c")
        s = lax.axis_index("s")
        w = c * NS + s

        @pl.when(s == 0)
        def _():
            pltpu.sync_copy(zeros_n_hbm, deg_sh)

        pltpu.sync_copy(col_hbm.at[pl.ds(w * e_per, e_per)], col_all)
        for k in range(K // 16):
            ones_buf[pl.ds(k * 16, 16)] = jnp.ones((16,), jnp.float32)
        plsc.subcore_barrier()

        @pl.loop(0, n_chunks)
        def _(i):
            off = i * K
            for k in range(K // 16):
                col_buf[pl.ds(k * 16, 16)] = col_all[pl.ds(off + k * 16, 16)]
            pltpu.sync_copy(ones_buf, deg_sh.at[col_buf], add=True)

        plsc.subcore_barrier()

        @pl.when(s == 0)
        def _():
            pltpu.sync_copy(deg_sh, degp_hbm.at[c, 0])

    return deg_kernel


@functools.lru_cache(maxsize=None)
def _make_scatter_kernel(E, N, H, with_q):
    e_per = E // NW
    n_chunks = e_per // K
    Np = -(-N // (8 * NS)) * (8 * NS)
    n_per = Np // NS

    if n_chunks % 2 != 1:
        raise ValueError("pipeline assumes an odd chunk count")
    n_pairs = (n_chunks - 1) // 2

    out_type = [jax.ShapeDtypeStruct((NC, Np, H), jnp.float32)]
    scratch = [
        pltpu.VMEM((e_per,), jnp.int32),
        pltpu.VMEM((e_per,), jnp.int32),
    ]
    for _ in range(2):
        scratch += [
            pltpu.VMEM((K,), jnp.int32),
            pltpu.VMEM((K,), jnp.int32),
            pltpu.VMEM((K, H), jnp.float32),
            pltpu.SemaphoreType.DMA,
            pltpu.SemaphoreType.DMA,
        ]
    scratch.append(pltpu.VMEM_SHARED((Np, H), jnp.float32))
    if with_q:
        out_type.append(jax.ShapeDtypeStruct((NC, 1, N), jnp.float32))
        for _ in range(2):
            scratch += [
                pltpu.VMEM((K,), jnp.float32),
                pltpu.SemaphoreType.DMA,
                pltpu.SemaphoreType.DMA,
            ]
        scratch.append(pltpu.VMEM_SHARED((N,), jnp.float32))

    def body(*refs):
        if with_q:
            (y_hbm, row_hbm, col_hbm, dis_hbm, zeros_nh, zeros_n,
             vout, qout,
             row_all, col_all,
             rbA, cbA, rvA, gsA, ssA,
             rbB, cbB, rvB, gsB, ssB,
             acc_sh,
             dvA, qgA, qsA, dvB, qgB, qsB, q_sh) = refs
            bufs = ((rbA, cbA, rvA, gsA, ssA, dvA, qgA, qsA),
                    (rbB, cbB, rvB, gsB, ssB, dvB, qgB, qsB))
        else:
            (y_hbm, row_hbm, col_hbm, zeros_nh,
             vout,
             row_all, col_all,
             rbA, cbA, rvA, gsA, ssA,
             rbB, cbB, rvB, gsB, ssB,
             acc_sh) = refs
            bufs = ((rbA, cbA, rvA, gsA, ssA, None, None, None),
                    (rbB, cbB, rvB, gsB, ssB, None, None, None))

        c = lax.axis_index("c")
        s = lax.axis_index("s")
        w = c * NS + s

        pltpu.sync_copy(zeros_nh.at[pl.ds(s * n_per, n_per)],
                        acc_sh.at[pl.ds(s * n_per, n_per)])
        if with_q:
            @pl.when(s == 0)
            def _():
                pltpu.sync_copy(zeros_n, q_sh)
        pltpu.sync_copy(row_hbm.at[pl.ds(w * e_per, e_per)], row_all)
        pltpu.sync_copy(col_hbm.at[pl.ds(w * e_per, e_per)], col_all)
        plsc.subcore_barrier()

        def prep(i, b):
            rb, cb = b[0], b[1]
            off = i * K
            for k in range(K // 16):
                rb[pl.ds(k * 16, 16)] = row_all[pl.ds(off + k * 16, 16)]
                cb[pl.ds(k * 16, 16)] = col_all[pl.ds(off + k * 16, 16)]

        def gstart(b):
            ds = [pltpu.async_copy(y_hbm.at[b[0]], b[2], b[3])]
            if with_q:
                ds.append(pltpu.async_copy(dis_hbm.at[b[1]], b[5], b[6]))
            return ds

        def sstart(b):
            ds = [pltpu.async_copy(b[2], acc_sh.at[b[1]], b[4], add=True)]
            if with_q:
                ds.append(pltpu.async_copy(b[5], q_sh.at[b[0]], b[7], add=True))
            return ds

        def wait(ds):
            for d in ds:
                d.wait()

        A, B = bufs

        @pl.loop(0, n_pairs)
        def _(i):
            ch = 2 * i
            prep(ch, A)
            gA = gstart(A)
            prep(ch + 1, B)
            gB = gstart(B)
            wait(gA)
            sA = sstart(A)
            wait(gB)
            sB = sstart(B)
            wait(sA)
            wait(sB)

        prep(n_chunks - 1, A)
        gA = gstart(A)
        wait(gA)
        wait(sstart(A))

        plsc.subcore_barrier()

        pltpu.sync_copy(acc_sh.at[pl.ds(s * n_per, n_per)],
                        vout.at[c, pl.ds(s * n_per, n_per)])
        if with_q:
            @pl.when(s == 0)
            def _():
                pltpu.sync_copy(q_sh, qout.at[c, 0])

    return pl.kernel(
        body,
        out_type=tuple(out_type) if with_q else out_type[0],
        mesh=_sc_mesh(),
        scratch_types=scratch,
    )


def _tc_dis_y1(degp, x, W1, BN=1000):
    N, D = x.shape
    H = W1.shape[1]

    def tc1(degp_ref, x_ref, w_ref, dis_ref, y_ref):
        dp = degp_ref[...]
        deg = dp[0] + dp[1]
        dis = jnp.where(deg > 0, lax.rsqrt(deg), 0.0)
        dis_ref[...] = dis
        xw = jnp.dot(x_ref[...], w_ref[...], preferred_element_type=jnp.float32)
        y_ref[...] = dis * xw

    return pl.pallas_call(
        tc1,
        grid=(N // BN,),
        in_specs=[
            pl.BlockSpec((NC, BN, 1), lambda i: (0, i, 0)),
            pl.BlockSpec((BN, D), lambda i: (i, 0)),
            pl.BlockSpec((D, H), lambda i: (0, 0)),
        ],
        out_specs=[
            pl.BlockSpec((BN, 1), lambda i: (i, 0)),
            pl.BlockSpec((BN, H), lambda i: (i, 0)),
        ],
        out_shape=[
            jax.ShapeDtypeStruct((N, 1), jnp.float32),
            jax.ShapeDtypeStruct((N, H), jnp.float32),
        ],
    )(degp, x, W1)


def _tc_mid(vp, dis, b, W, BN=1000):
    _, _, H = vp.shape
    N = dis.shape[0]

    def tc2(vp_ref, dis_ref, b_ref, w_ref, y_ref):
        v = vp_ref[0] + vp_ref[1]
        dis = dis_ref[...]
        h = dis * v + b_ref[...]
        y_ref[...] = dis * jnp.dot(
            h, w_ref[...], preferred_element_type=jnp.float32)

    return pl.pallas_call(
        tc2,
        grid=(N // BN,),
        in_specs=[
            pl.BlockSpec((NC, BN, H), lambda i: (0, i, 0)),
            pl.BlockSpec((BN, 1), lambda i: (i, 0)),
            pl.BlockSpec((H,), lambda i: (0,)),
            pl.BlockSpec((H, H), lambda i: (0, 0)),
        ],
        out_specs=pl.BlockSpec((BN, H), lambda i: (i, 0)),
        out_shape=jax.ShapeDtypeStruct((N, H), jnp.float32),
    )(vp, dis, b, W)


def _tc_final(vp, qp, dis, b2, W3, Wl, b3, bl, BN=1000):
    _, _, H = vp.shape
    N = dis.shape[0]
    nsteps = N // BN

    def tc3(vp_ref, qp_ref, dis_ref, b2_ref, w3_ref, wl_ref, b3_ref, bl_ref,
            out_ref, sacc):
        i = pl.program_id(0)

        @pl.when(i == 0)
        def _():
            sacc[...] = jnp.zeros_like(sacc)

        v = vp_ref[0] + vp_ref[1]
        dis = dis_ref[...]
        h2 = dis * v + b2_ref[...]
        g = dis * (qp_ref[0] + qp_ref[1])
        m = jnp.dot(h2, w3_ref[...], preferred_element_type=jnp.float32)
        sacc[...] += jnp.sum(g * m, axis=0, keepdims=True)

        @pl.when(i == nsteps - 1)
        def _():
            pooled = sacc[...] * (1.0 / N) + b3_ref[...][None, :]
            out_ref[...] = jnp.dot(pooled, wl_ref[...],
                                   preferred_element_type=jnp.float32) + bl_ref[...]

    return pl.pallas_call(
        tc3,
        grid=(nsteps,),
        in_specs=[
            pl.BlockSpec((NC, BN, H), lambda i: (0, i, 0)),
            pl.BlockSpec((NC, BN, 1), lambda i: (0, i, 0)),
            pl.BlockSpec((BN, 1), lambda i: (i, 0)),
            pl.BlockSpec((H,), lambda i: (0,)),
            pl.BlockSpec((H, H), lambda i: (0, 0)),
            pl.BlockSpec((H, 1), lambda i: (0, 0)),
            pl.BlockSpec((H,), lambda i: (0,)),
            pl.BlockSpec((1,), lambda i: (0,)),
        ],
        out_specs=pl.BlockSpec((1, 1), lambda i: (0, 0)),
        out_shape=jax.ShapeDtypeStruct((1, 1), jnp.float32),
        scratch_shapes=[pltpu.VMEM((1, H), jnp.float32)],
    )(vp, qp, dis, b2, W3, Wl, b3, bl)


def kernel(x, edge_index, edge_attr, W1, b1, W2, b2, W3, b3, Wl, bl):
    N, _ = x.shape
    H = W1.shape[1]
    E = edge_index.shape[1]
    row = edge_index[0]
    col = edge_index[1]
    Np = -(-N // (8 * NS)) * (8 * NS)
    zeros_nh = jnp.zeros((Np, H), jnp.float32)
    zeros_n = jnp.zeros((N,), jnp.float32)

    degp = _make_deg_kernel(E, N)(col, zeros_n)
    dis2, y1 = _tc_dis_y1(degp.reshape(NC, N, 1), x, W1)
    dis1 = dis2.reshape(N)
    v1p, qp = _make_scatter_kernel(E, N, H, True)(
        y1, row, col, dis1, zeros_nh, zeros_n)
    y2 = _tc_mid(v1p, dis2, b1, W2)
    v2p = _make_scatter_kernel(E, N, H, False)(y2, row, col, zeros_nh)
    return _tc_final(v2p, qp.reshape(NC, N, 1), dis2, b2, W3, Wl, b3, bl)
